# Initial kernel scaffold; baseline (speedup 1.0000x reference)
import jax
import jax.numpy as jnp
from jax.experimental import pallas as pl


def kernel(node_repr, adj_list, weight, mask, W):
    raise NotImplementedError("write your pallas kernel here")


# speedup vs baseline: 1.7700x; 1.7700x over previous
import functools

import jax
import jax.numpy as jnp
from jax import lax
from jax.experimental import pallas as pl
from jax.experimental.pallas import tpu as pltpu
from jax.experimental.pallas import tpu_sc as plsc

EPS = 1e-10

NC = 2
NS = 16
NW = NC * NS

D = 128
K = 32
P = 320
N_PAD = NW * P
C = 4
ROWS = C * K
CHUNKS = P // C
LANES = 16
FV = D // LANES


def _row_block(i):
    return (i, 0)


def _w_block(i):
    return (0, 0)


def _norm(v):
    n = jnp.sqrt(jnp.sum(v * v, axis=-1, keepdims=True))
    return jnp.maximum(n, EPS)


def _exp_map_zero(v):
    n = _norm(v)
    return jnp.tanh(n) * v / n


def _log_map_zero(y):
    n = _norm(y)
    c = jnp.clip(n, -1.0 + 1e-7, 1.0 - 1e-7)
    artanh = 0.5 * jnp.log((1.0 + c) / (1.0 - c))
    return artanh * y / n


def _dot(x, w):
    return jnp.dot(x, w, preferred_element_type=jnp.float32,
                   precision=lax.Precision.HIGHEST)


def _tc_first(x, mask, W, n, blk, grid):
    def body(x_ref, m_ref, w_ref, o_ref):
        m = m_ref[...]
        o_ref[...] = _dot(x_ref[...] * m, w_ref[...]) * m

    return pl.pallas_call(
        body,
        grid=(grid,),
        in_specs=[pl.BlockSpec((blk, D), _row_block),
                  pl.BlockSpec((blk, 1), _row_block),
                  pl.BlockSpec((D, D), _w_block)],
        out_specs=pl.BlockSpec((blk, D), _row_block),
        out_shape=jax.ShapeDtypeStruct((n, D), jnp.float32),
    )(x, mask, W)


def _tc_mid(comb, mask, W, n, blk, grid):
    def body(c_ref, m_ref, w_ref, o_ref):
        m = m_ref[...]
        x = _exp_map_zero(c_ref[...] * m) * m
        x = jax.nn.relu(x) * m
        x = _log_map_zero(x) * m
        o_ref[...] = _dot(x, w_ref[...]) * m

    return pl.pallas_call(
        body,
        grid=(grid,),
        in_specs=[pl.BlockSpec((blk, D), _row_block),
                  pl.BlockSpec((blk, 1), _row_block),
                  pl.BlockSpec((D, D), _w_block)],
        out_specs=pl.BlockSpec((blk, D), _row_block),
        out_shape=jax.ShapeDtypeStruct((n, D), jnp.float32),
    )(comb, mask, W)


def _tc_final(comb, mask, n, blk, grid):
    def body(c_ref, m_ref, o_ref):
        m = m_ref[...]
        x = _exp_map_zero(c_ref[...] * m) * m
        o_ref[...] = jax.nn.relu(x) * m

    return pl.pallas_call(
        body,
        grid=(grid,),
        in_specs=[pl.BlockSpec((blk, D), _row_block),
                  pl.BlockSpec((blk, 1), _row_block)],
        out_specs=pl.BlockSpec((blk, D), _row_block),
        out_shape=jax.ShapeDtypeStruct((n, D), jnp.float32),
    )(comb, mask)


def _splat(wv, idx16):
    return lax.gather(
        wv, idx16.reshape(LANES, 1),
        lax.GatherDimensionNumbers(offset_dims=(), collapsed_slice_dims=(0,),
                                   start_index_map=(0,)),
        slice_sizes=(1,), mode=lax.GatherScatterMode.PROMISE_IN_BOUNDS)


def _sc_compute_chunk(c, g, w_all, out_all, splat_idx):
    for j in range(C):
        row = c * C + j
        wv = [w_all[row, pl.ds(h * LANES, LANES)] for h in range(K // LANES)]
        acc = [jnp.zeros((LANES,), jnp.float32) for _ in range(FV)]
        for k in range(K):
            wk = _splat(wv[k // LANES], splat_idx[k % LANES])
            r = j * K + k
            for f in range(FV):
                acc[f] = acc[f] + wk * g[r, pl.ds(f * LANES, LANES)]
        for f in range(FV):
            out_all[row, pl.ds(f * LANES, LANES)] = acc[f]


def _sc_aggregate(msg, adj2d, w_pad):
    mesh = plsc.VectorSubcoreMesh(core_axis_name="c", subcore_axis_name="s",
                                  num_cores=NC, num_subcores=NS)

    @functools.partial(
        pl.kernel,
        out_type=jax.ShapeDtypeStruct((N_PAD, D), jnp.float32),
        mesh=mesh,
        scratch_types=[
            pltpu.VMEM((CHUNKS, ROWS), jnp.int32),
            pltpu.VMEM((P, K), jnp.float32),
            pltpu.VMEM((P, D), jnp.float32),
            pltpu.VMEM((ROWS, D), jnp.float32),
            pltpu.VMEM((ROWS, D), jnp.float32),
            pltpu.SemaphoreType.DMA,
            pltpu.SemaphoreType.DMA,
        ],
    )
    def agg(msg_hbm, adj_hbm, w_hbm, out_hbm,
            idx_all, w_all, out_all, g0, g1, sem0, sem1):
        wid = lax.axis_index("s") * NC + lax.axis_index("c")
        splat_idx = [jnp.full((LANES,), i, jnp.int32) for i in range(LANES)]
        pltpu.sync_copy(adj_hbm.at[pl.ds(wid * CHUNKS, CHUNKS)], idx_all)
        pltpu.sync_copy(w_hbm.at[pl.ds(wid * P, P)], w_all)
        pltpu.async_copy(msg_hbm.at[idx_all.at[0]], g0, sem0)

        @pl.loop(0, CHUNKS // 2)
        def _pair(t):
            c0 = t * 2
            c1 = c0 + 1
            pltpu.make_async_copy(msg_hbm.at[idx_all.at[c0]], g0, sem0).wait()
            pltpu.async_copy(msg_hbm.at[idx_all.at[c1]], g1, sem1)
            _sc_compute_chunk(c0, g0, w_all, out_all, splat_idx)
            pltpu.make_async_copy(msg_hbm.at[idx_all.at[c1]], g1, sem1).wait()

            @pl.when(t + 1 < CHUNKS // 2)
            def _():
                pltpu.async_copy(msg_hbm.at[idx_all.at[c0 + 2]], g0, sem0)

            _sc_compute_chunk(c1, g1, w_all, out_all, splat_idx)

        pltpu.sync_copy(out_all, out_hbm.at[pl.ds(wid * P, P)])

    return agg(msg, adj2d, w_pad)


def kernel(node_repr, adj_list, weight, mask, W):
    n, k = adj_list.shape
    blk = 1000
    grid = n // blk

    adj = adj_list.astype(jnp.int32)
    pad = N_PAD - n
    adj_pad = jnp.pad(adj, ((0, pad), (0, 0)))
    w_pad = jnp.pad(weight, ((0, pad), (0, 0)))
    adj2d = adj_pad.reshape(N_PAD * K // ROWS, ROWS)

    msg0 = _tc_first(node_repr, mask, W, n, blk, grid)
    comb0 = _sc_aggregate(msg0, adj2d, w_pad)[:n]
    msg1 = _tc_mid(comb0, mask, W, n, blk, grid)
    comb1 = _sc_aggregate(msg1, adj2d, w_pad)[:n]
    return _tc_final(comb1, mask, n, blk, grid)


# speedup vs baseline: 1.9197x; 1.0846x over previous
import functools

import jax
import jax.numpy as jnp
from jax import lax
from jax.experimental import pallas as pl
from jax.experimental.pallas import tpu as pltpu
from jax.experimental.pallas import tpu_sc as plsc

EPS = 1e-10

NC = 2
NS = 16
NW = NC * NS

D = 128
K = 32
P = 320
N_PAD = NW * P
C = 4
ROWS = C * K
CHUNKS = P // C
LANES = 16
FV = D // LANES


def _row_block(i):
    return (i, 0)


def _w_block(i):
    return (0, 0)


def _norm(v):
    n = jnp.sqrt(jnp.sum(v * v, axis=-1, keepdims=True))
    return jnp.maximum(n, EPS)


def _exp_map_zero(v):
    n = _norm(v)
    return jnp.tanh(n) * v / n


def _log_map_zero(y):
    n = _norm(y)
    c = jnp.clip(n, -1.0 + 1e-7, 1.0 - 1e-7)
    artanh = 0.5 * jnp.log((1.0 + c) / (1.0 - c))
    return artanh * y / n


def _dot(x, w):
    return jnp.dot(x, w, preferred_element_type=jnp.float32,
                   precision=lax.Precision.HIGHEST)


def _tc_first(x, mask, W, n, blk, grid):
    def body(x_ref, m_ref, w_ref, o_ref):
        m = m_ref[...]
        o_ref[...] = _dot(x_ref[...] * m, w_ref[...]) * m

    return pl.pallas_call(
        body,
        grid=(grid,),
        in_specs=[pl.BlockSpec((blk, D), _row_block),
                  pl.BlockSpec((blk, 1), _row_block),
                  pl.BlockSpec((D, D), _w_block)],
        out_specs=pl.BlockSpec((blk, D), _row_block),
        out_shape=jax.ShapeDtypeStruct((n, D), jnp.float32),
    )(x, mask, W)


def _tc_mid(comb, mask, W, n, blk, grid):
    def body(c_ref, m_ref, w_ref, o_ref):
        m = m_ref[...]
        x = _exp_map_zero(c_ref[...] * m) * m
        x = jax.nn.relu(x) * m
        x = _log_map_zero(x) * m
        o_ref[...] = _dot(x, w_ref[...]) * m

    return pl.pallas_call(
        body,
        grid=(grid,),
        in_specs=[pl.BlockSpec((blk, D), _row_block),
                  pl.BlockSpec((blk, 1), _row_block),
                  pl.BlockSpec((D, D), _w_block)],
        out_specs=pl.BlockSpec((blk, D), _row_block),
        out_shape=jax.ShapeDtypeStruct((n, D), jnp.float32),
    )(comb, mask, W)


def _tc_final(comb, mask, n, blk, grid):
    def body(c_ref, m_ref, o_ref):
        m = m_ref[...]
        x = _exp_map_zero(c_ref[...] * m) * m
        o_ref[...] = jax.nn.relu(x) * m

    return pl.pallas_call(
        body,
        grid=(grid,),
        in_specs=[pl.BlockSpec((blk, D), _row_block),
                  pl.BlockSpec((blk, 1), _row_block)],
        out_specs=pl.BlockSpec((blk, D), _row_block),
        out_shape=jax.ShapeDtypeStruct((n, D), jnp.float32),
    )(comb, mask)


def _splat(wv, idx16):
    return lax.gather(
        wv, idx16.reshape(LANES, 1),
        lax.GatherDimensionNumbers(offset_dims=(), collapsed_slice_dims=(0,),
                                   start_index_map=(0,)),
        slice_sizes=(1,), mode=lax.GatherScatterMode.PROMISE_IN_BOUNDS)


def _sc_compute_chunk(c, g, w_all, ostage, orow, splat_idx):
    for j in range(C):
        row = c * C + j
        wv = [w_all[row, pl.ds(h * LANES, LANES)] for h in range(K // LANES)]
        acc = [jnp.zeros((LANES,), jnp.float32) for _ in range(FV)]
        for k in range(K):
            wk = _splat(wv[k // LANES], splat_idx[k % LANES])
            r = j * K + k
            for f in range(FV):
                acc[f] = acc[f] + wk * g[r, pl.ds(f * LANES, LANES)]
        for f in range(FV):
            ostage[orow + j, pl.ds(f * LANES, LANES)] = acc[f]


def _sc_aggregate(msg, adj2d, w_pad):
    mesh = plsc.VectorSubcoreMesh(core_axis_name="c", subcore_axis_name="s",
                                  num_cores=NC, num_subcores=NS)

    @functools.partial(
        pl.kernel,
        out_type=jax.ShapeDtypeStruct((N_PAD, D), jnp.float32),
        mesh=mesh,
        scratch_types=[
            pltpu.VMEM((CHUNKS, ROWS), jnp.int32),
            pltpu.VMEM((P, K), jnp.float32),
            pltpu.VMEM((2 * C, D), jnp.float32),
            pltpu.VMEM((2 * C, D), jnp.float32),
            pltpu.VMEM((ROWS, D), jnp.float32),
            pltpu.VMEM((ROWS, D), jnp.float32),
            pltpu.VMEM((ROWS, D), jnp.float32),
            pltpu.VMEM((ROWS, D), jnp.float32),
            pltpu.SemaphoreType.DMA,
            pltpu.SemaphoreType.DMA,
            pltpu.SemaphoreType.DMA,
            pltpu.SemaphoreType.DMA,
            pltpu.SemaphoreType.DMA,
            pltpu.SemaphoreType.DMA,
        ],
    )
    def agg(msg_hbm, adj_hbm, w_hbm, out_hbm,
            idx_all, w_all, os0, os1, g0, g1, g2, g3,
            sem0, sem1, sem2, sem3, osem0, osem1):
        wid = lax.axis_index("s") * NC + lax.axis_index("c")
        splat_idx = [jnp.full((LANES,), i, jnp.int32) for i in range(LANES)]
        gbuf = [g0, g1, g2, g3]
        sem = [sem0, sem1, sem2, sem3]
        ostage = [os0, os1]
        osem = [osem0, osem1]
        nb = len(gbuf)
        pltpu.sync_copy(adj_hbm.at[pl.ds(wid * CHUNKS, CHUNKS)], idx_all)
        pltpu.sync_copy(w_hbm.at[pl.ds(wid * P, P)], w_all)
        for b in range(nb):
            pltpu.async_copy(msg_hbm.at[idx_all.at[b]], gbuf[b], sem[b])

        @pl.loop(0, CHUNKS // nb)
        def _quad(t):
            base = t * nb

            @pl.when(t >= 1)
            def _():
                for s in range(2):
                    pltpu.make_async_copy(
                        ostage[s],
                        out_hbm.at[pl.ds(0, 2 * C)],
                        osem[s]).wait()

            for b in range(nb):
                c = base + b
                pltpu.make_async_copy(msg_hbm.at[idx_all.at[c]],
                                      gbuf[b], sem[b]).wait()
                s = b // 2
                _sc_compute_chunk(c, gbuf[b], w_all, ostage[s],
                                  (b % 2) * C, splat_idx)

                @pl.when(c + nb < CHUNKS)
                def _():
                    pltpu.async_copy(msg_hbm.at[idx_all.at[c + nb]],
                                     gbuf[b], sem[b])

                if b % 2 == 1:
                    pltpu.async_copy(
                        ostage[s],
                        out_hbm.at[pl.ds(wid * P + (c - 1) * C, 2 * C)],
                        osem[s])

        for s in range(2):
            pltpu.make_async_copy(ostage[s], out_hbm.at[pl.ds(0, 2 * C)],
                                  osem[s]).wait()

    return agg(msg, adj2d, w_pad)


def kernel(node_repr, adj_list, weight, mask, W):
    n, k = adj_list.shape
    blk = 1000
    grid = n // blk

    adj = adj_list.astype(jnp.int32)
    pad = N_PAD - n
    adj_pad = jnp.pad(adj, ((0, pad), (0, 0)))
    w_pad = jnp.pad(weight, ((0, pad), (0, 0)))
    adj2d = adj_pad.reshape(N_PAD * K // ROWS, ROWS)

    msg0 = _tc_first(node_repr, mask, W, n, blk, grid)
    comb0 = _sc_aggregate(msg0, adj2d, w_pad)[:n]
    msg1 = _tc_mid(comb0, mask, W, n, blk, grid)
    comb1 = _sc_aggregate(msg1, adj2d, w_pad)[:n]
    return _tc_final(comb1, mask, n, blk, grid)


# speedup vs baseline: 4.4536x; 2.3199x over previous
import functools

import jax
import jax.numpy as jnp
from jax import lax
from jax.experimental import pallas as pl
from jax.experimental.pallas import tpu as pltpu
from jax.experimental.pallas import tpu_sc as plsc

EPS = 1e-10

NC = 2
NS = 16
NW = NC * NS

D = 128
DW = D // 2
K = 32
P = 320
N_PAD = NW * P
C = 4
ROWS = C * K
CHUNKS = P // C
HALF = CHUNKS // 2
LANES = 16
FV = D // LANES
G32 = D // 32
FILL = N_PAD // NS


def _row_block(i):
    return (i, 0)


def _w_block(i):
    return (0, 0)


def _norm(v):
    n = jnp.sqrt(jnp.sum(v * v, axis=-1, keepdims=True))
    return jnp.maximum(n, EPS)


def _exp_map_zero(v):
    n = _norm(v)
    return jnp.tanh(n) * v / n


def _log_map_zero(y):
    n = _norm(y)
    c = jnp.clip(n, -1.0 + 1e-7, 1.0 - 1e-7)
    artanh = 0.5 * jnp.log((1.0 + c) / (1.0 - c))
    return artanh * y / n


def _dot(x, w):
    return jnp.dot(x, w, preferred_element_type=jnp.float32,
                   precision=lax.Precision.HIGHEST)


def _tc_first(x, mask, W, n, blk, grid):
    def body(x_ref, m_ref, w_ref, o_ref):
        m = m_ref[...]
        o_ref[...] = _dot(x_ref[...] * m, w_ref[...]) * m

    return pl.pallas_call(
        body,
        grid=(grid,),
        in_specs=[pl.BlockSpec((blk, D), _row_block),
                  pl.BlockSpec((blk, 1), _row_block),
                  pl.BlockSpec((D, D), _w_block)],
        out_specs=pl.BlockSpec((blk, D), _row_block),
        out_shape=jax.ShapeDtypeStruct((n, D), jnp.float32),
    )(x, mask, W)


def _tc_mid(comb, mask, W, n, blk, grid):
    def body(c_ref, m_ref, w_ref, o_ref):
        m = m_ref[...]
        x = _exp_map_zero(c_ref[...] * m) * m
        x = jax.nn.relu(x) * m
        x = _log_map_zero(x) * m
        o_ref[...] = _dot(x, w_ref[...]) * m

    return pl.pallas_call(
        body,
        grid=(grid,),
        in_specs=[pl.BlockSpec((blk, D), _row_block),
                  pl.BlockSpec((blk, 1), _row_block),
                  pl.BlockSpec((D, D), _w_block)],
        out_specs=pl.BlockSpec((blk, D), _row_block),
        out_shape=jax.ShapeDtypeStruct((n, D), jnp.float32),
    )(comb, mask, W)


def _tc_final(comb, mask, n, blk, grid):
    def body(c_ref, m_ref, o_ref):
        m = m_ref[...]
        x = _exp_map_zero(c_ref[...] * m) * m
        o_ref[...] = jax.nn.relu(x) * m

    return pl.pallas_call(
        body,
        grid=(grid,),
        in_specs=[pl.BlockSpec((blk, D), _row_block),
                  pl.BlockSpec((blk, 1), _row_block)],
        out_specs=pl.BlockSpec((blk, D), _row_block),
        out_shape=jax.ShapeDtypeStruct((n, D), jnp.float32),
    )(comb, mask)


def _pack_msg_i32(msg):
    n = msg.shape[0]
    m4 = msg.reshape(n, G32, 2, LANES).transpose(0, 1, 3, 2).reshape(n, D)
    return lax.bitcast_convert_type(
        m4.astype(jnp.bfloat16).reshape(n, DW, 2), jnp.int32)


def _splat(wv, idx16):
    return lax.gather(
        wv, idx16.reshape(LANES, 1),
        lax.GatherDimensionNumbers(offset_dims=(), collapsed_slice_dims=(0,),
                                   start_index_map=(0,)),
        slice_sizes=(1,), mode=lax.GatherScatterMode.PROMISE_IN_BOUNDS)


def _sc_compute_chunk(c, g, w_all, out_all, obase, splat_idx, himask):
    for j in range(C):
        row = c * C + j
        wv = [w_all[row, pl.ds(h * LANES, LANES)] for h in range(K // LANES)]
        acc = [jnp.zeros((LANES,), jnp.float32) for _ in range(FV)]
        for k in range(K):
            wk = _splat(wv[k // LANES], splat_idx[k % LANES])
            r = j * K + k
            for g4 in range(G32):
                w32 = g[r, pl.ds(g4 * LANES, LANES)]
                lo = lax.bitcast_convert_type(w32 << 16, jnp.float32)
                hi = lax.bitcast_convert_type(w32 & himask, jnp.float32)
                acc[2 * g4] = acc[2 * g4] + wk * lo
                acc[2 * g4 + 1] = acc[2 * g4 + 1] + wk * hi
        for f in range(FV):
            out_all[obase + j, pl.ds(f * LANES, LANES)] = acc[f]


def _sc_aggregate(msg_i, adj2d, w_pad):
    mesh = plsc.VectorSubcoreMesh(core_axis_name="c", subcore_axis_name="s",
                                  num_cores=NC, num_subcores=NS)

    @functools.partial(
        pl.kernel,
        out_type=jax.ShapeDtypeStruct((N_PAD, D), jnp.float32),
        mesh=mesh,
        scratch_types=[
            pltpu.VMEM_SHARED((N_PAD, DW), jnp.int32),
            pltpu.VMEM((HALF, ROWS), jnp.int32),
            pltpu.VMEM((P, K), jnp.float32),
            pltpu.VMEM((HALF * C // 2, D), jnp.float32),
            pltpu.VMEM((ROWS, DW), jnp.int32),
            pltpu.VMEM((ROWS, DW), jnp.int32),
            pltpu.SemaphoreType.DMA,
            pltpu.SemaphoreType.DMA,
            pltpu.SemaphoreType.DMA,
        ],
    )
    def agg(msg_hbm, adj_hbm, w_hbm, out_hbm,
            table, idx_all, w_all, out_all, g0, g1, sem0, sem1, fsem):
        cid = lax.axis_index("c")
        sid = lax.axis_index("s")
        wid = sid * NC + cid
        splat_idx = [jnp.full((LANES,), i, jnp.int32) for i in range(LANES)]
        himask = jnp.full((LANES,), -65536, jnp.int32)
        gbuf = [g0, g1]
        sem = [sem0, sem1]

        pltpu.async_copy(msg_hbm.at[pl.ds(sid * FILL, FILL)],
                         table.at[pl.ds(sid * FILL, FILL)], fsem)
        pltpu.sync_copy(w_hbm.at[pl.ds(wid * P, P)], w_all)
        pltpu.make_async_copy(msg_hbm.at[pl.ds(sid * FILL, FILL)],
                              table.at[pl.ds(sid * FILL, FILL)], fsem).wait()
        plsc.subcore_barrier()

        QC = HALF // 2

        @pl.loop(0, 4)
        def _quarter(qp):
            half = qp // 2

            @pl.when(qp % 2 == 0)
            def _():
                pltpu.sync_copy(
                    adj_hbm.at[pl.ds(wid * CHUNKS + half * HALF, HALF)],
                    idx_all)

            qbase = (qp % 2) * QC
            for b in range(2):
                pltpu.async_copy(table.at[idx_all.at[qbase + b]],
                                 gbuf[b], sem[b])

            @pl.loop(0, QC // 2)
            def _pair(t):
                for b in range(2):
                    ci = qbase + t * 2 + b
                    c = qp * QC + t * 2 + b
                    pltpu.make_async_copy(table.at[idx_all.at[ci]],
                                          gbuf[b], sem[b]).wait()

                    @pl.when(t * 2 + b + 2 < QC)
                    def _():
                        pltpu.async_copy(table.at[idx_all.at[ci + 2]],
                                         gbuf[b], sem[b])

                    _sc_compute_chunk(c, gbuf[b], w_all, out_all,
                                      (t * 2 + b) * C, splat_idx, himask)

            pltpu.sync_copy(
                out_all,
                out_hbm.at[pl.ds(wid * P + qp * QC * C, QC * C)])

    return agg(msg_i, adj2d, w_pad)


def kernel(node_repr, adj_list, weight, mask, W):
    n, _ = adj_list.shape
    blk = 1024
    grid = N_PAD // blk

    pad = N_PAD - n
    x_pad = jnp.pad(node_repr, ((0, pad), (0, 0)))
    m_pad = jnp.pad(mask, ((0, pad), (0, 0)))
    adj_pad = jnp.pad(adj_list.astype(jnp.int32), ((0, pad), (0, 0)))
    w_pad = jnp.pad(weight, ((0, pad), (0, 0)))
    adj2d = adj_pad.reshape(N_PAD * K // ROWS, ROWS)

    msg0 = _tc_first(x_pad, m_pad, W, N_PAD, blk, grid)
    comb0 = _sc_aggregate(_pack_msg_i32(msg0), adj2d, w_pad)
    msg1 = _tc_mid(comb0, m_pad, W, N_PAD, blk, grid)
    comb1 = _sc_aggregate(_pack_msg_i32(msg1), adj2d, w_pad)
    return _tc_final(comb1, m_pad, N_PAD, blk, grid)[:n]
